# Initial kernel scaffold; baseline (speedup 1.0000x reference)
import jax
import jax.numpy as jnp
from jax.experimental import pallas as pl


def kernel(x, sims, mask, ln_w, ln_b, q_w, k_w, v_w, indices, labels, num_spixels):
    raise NotImplementedError("write your pallas kernel here")


# speedup vs baseline: 1.0109x; 1.0109x over previous
import functools

import jax
import jax.numpy as jnp
from jax import lax
from jax.experimental import pallas as pl

B, C, H, W = 2, 96, 384, 384
QK_DIM = 96
NUM_HEADS = 3
K_SP = 576
TOPK = 64
HEAD_DIM = QK_DIM // NUM_HEADS
SCALE_F = HEAD_DIM ** (-0.5)
HW = H * W
NTOK = B * K_SP * TOPK

TP = 2048
G_SP = 4


def _ln_body(x_ref, o_ref):
    xb = x_ref[0]
    u = jnp.mean(xb, axis=0, keepdims=True)
    d = xb - u
    s = jnp.mean(d * d, axis=0, keepdims=True)
    xn = d * lax.rsqrt(s + 1e-6)
    o_ref[0] = xn.T


def _ln_transpose(x3):
    return pl.pallas_call(
        _ln_body,
        grid=(B, HW // TP),
        in_specs=[pl.BlockSpec((1, C, TP), lambda b, i: (b, 0, i))],
        out_specs=pl.BlockSpec((1, TP, C), lambda b, i: (b, i, 0)),
        out_shape=jax.ShapeDtypeStruct((B, HW, C), jnp.float32),
    )(x3)


def _attn_body(xg_ref, sims_ref, mask_ref, qw_ref, kw_ref, vw_ref, bias_ref, o_ref):
    xb = xg_ref[...]
    q = jnp.dot(xb, qw_ref[...], preferred_element_type=jnp.float32) + bias_ref[0]
    k = jnp.dot(xb, kw_ref[...], preferred_element_type=jnp.float32) + bias_ref[1]
    v = jnp.dot(xb, vw_ref[...], preferred_element_type=jnp.float32) + bias_ref[2]
    for g in range(G_SP):
        sl = slice(g * TOPK, (g + 1) * TOPK)
        qg, kg, vg = q[sl], k[sl], v[sl]
        sims_g = sims_ref[0, g][:, None]
        mask_g = mask_ref[0, g][None, :]
        outs = []
        for h in range(NUM_HEADS):
            hs = slice(h * HEAD_DIM, (h + 1) * HEAD_DIM)
            qh, kh, vh = qg[:, hs], kg[:, hs], vg[:, hs]
            qq = jnp.sum(qh * qh, axis=1, keepdims=True)
            kk = jnp.sum(kh * kh, axis=1, keepdims=True)
            gr = lax.dot_general(qh, kh, (((1,), (1,)), ((), ())),
                                 preferred_element_type=jnp.float32)
            d2 = qq + kk.T - 2.0 * gr
            dist = jnp.sqrt(jnp.maximum(d2, 1e-12))
            a = -SCALE_F * dist
            a = jnp.where(mask_g > 0, a, -1e9)
            a = a - jnp.max(a, axis=1, keepdims=True)
            e = jnp.exp(a)
            p = e / jnp.sum(e, axis=1, keepdims=True)
            vw_tok = sims_g * vh
            outs.append(jnp.dot(p, vw_tok, preferred_element_type=jnp.float32))
        o_ref[sl, :] = sims_g * jnp.concatenate(outs, axis=1)


def _attention(xg, sims2, mask2, qw_t, kw_t, vw_t, bias):
    ntile = TOPK * G_SP
    return pl.pallas_call(
        _attn_body,
        grid=(NTOK // ntile,),
        in_specs=[
            pl.BlockSpec((ntile, C), lambda i: (i, 0)),
            pl.BlockSpec((1, G_SP, TOPK), lambda i: (i, 0, 0)),
            pl.BlockSpec((1, G_SP, TOPK), lambda i: (i, 0, 0)),
            pl.BlockSpec((C, C), lambda i: (0, 0)),
            pl.BlockSpec((C, C), lambda i: (0, 0)),
            pl.BlockSpec((C, C), lambda i: (0, 0)),
            pl.BlockSpec((8, C), lambda i: (0, 0)),
        ],
        out_specs=pl.BlockSpec((ntile, C), lambda i: (i, 0)),
        out_shape=jax.ShapeDtypeStruct((NTOK, C), jnp.float32),
    )(xg, sims2, mask2, qw_t, kw_t, vw_t, bias)


def _merge_body(xn_ref, acc_ref, cnt_ref, vw_ref, vb_ref, o_ref):
    xb = xn_ref[0]
    v = jnp.dot(xb, vw_ref[...], preferred_element_type=jnp.float32) + vb_ref[0]
    cnt = cnt_ref[0]
    mean = acc_ref[0] / jnp.maximum(cnt, 1.0)
    sel = jnp.where(cnt >= 0.5, mean, v)
    o_ref[0] = sel.T


def _merge(xn_t, acc, cnt3, vw_t, vb):
    return pl.pallas_call(
        _merge_body,
        grid=(B, HW // TP),
        in_specs=[
            pl.BlockSpec((1, TP, C), lambda b, i: (b, i, 0)),
            pl.BlockSpec((1, TP, C), lambda b, i: (b, i, 0)),
            pl.BlockSpec((1, TP, 1), lambda b, i: (b, i, 0)),
            pl.BlockSpec((C, C), lambda b, i: (0, 0)),
            pl.BlockSpec((1, C), lambda b, i: (0, 0)),
        ],
        out_specs=pl.BlockSpec((1, C, TP), lambda b, i: (b, 0, i)),
        out_shape=jax.ShapeDtypeStruct((B, C, HW), jnp.float32),
    )(xn_t, acc, cnt3, vw_t, vb)


def kernel(x, sims, mask, ln_w, ln_b, q_w, k_w, v_w, indices, labels, num_spixels):
    qw_t = (q_w * ln_w[None, :]).T
    kw_t = (k_w * ln_w[None, :]).T
    vw_t = (v_w * ln_w[None, :]).T
    qb = q_w @ ln_b
    kb = k_w @ ln_b
    vb = v_w @ ln_b
    bias = jnp.zeros((8, C), jnp.float32).at[0].set(qb).at[1].set(kb).at[2].set(vb)

    x3 = x.reshape(B, C, HW)
    xn_t = _ln_transpose(x3)

    idx_g = (indices.reshape(B, K_SP * TOPK)
             + (jnp.arange(B, dtype=jnp.int32) * HW)[:, None]).reshape(-1)
    xn_flat = xn_t.reshape(B * HW, C)
    xg = jnp.take(xn_flat, idx_g, axis=0)

    sims2 = sims.reshape(B * K_SP // G_SP, G_SP, TOPK)
    mask2 = mask.reshape(B * K_SP // G_SP, G_SP, TOPK)
    out_tok = _attention(xg, sims2, mask2, qw_t, kw_t, vw_t, bias)

    acc = jax.ops.segment_sum(out_tok, idx_g, num_segments=B * HW)
    cnt = jax.ops.segment_sum(jnp.ones((NTOK,), jnp.float32), idx_g,
                              num_segments=B * HW)
    acc = acc.reshape(B, HW, C)
    cnt3 = cnt.reshape(B, HW, 1)

    out = _merge(xn_t, acc, cnt3, vw_t, vb[None, :])
    return out.reshape(B, C, H, W)


# speedup vs baseline: 1.4035x; 1.3884x over previous
import functools

import jax
import jax.numpy as jnp
from jax import lax
from jax.experimental import pallas as pl

B, C, H, W = 2, 96, 384, 384
QK_DIM = 96
NUM_HEADS = 3
K_SP = 576
TOPK = 64
HEAD_DIM = QK_DIM // NUM_HEADS
SCALE_F = HEAD_DIM ** (-0.5)
HW = H * W
NTOK = B * K_SP * TOPK

TP = 2048
G_SP = 4


def _ln_body(x_ref, o_ref):
    xb = x_ref[0]
    u = jnp.mean(xb, axis=0, keepdims=True)
    d = xb - u
    s = jnp.mean(d * d, axis=0, keepdims=True)
    xn = d * lax.rsqrt(s + 1e-6)
    o_ref[0] = xn.T


def _ln_transpose(x3):
    return pl.pallas_call(
        _ln_body,
        grid=(B, HW // TP),
        in_specs=[pl.BlockSpec((1, C, TP), lambda b, i: (b, 0, i))],
        out_specs=pl.BlockSpec((1, TP, C), lambda b, i: (b, i, 0)),
        out_shape=jax.ShapeDtypeStruct((B, HW, C), jnp.float32),
    )(x3)


def _attn_body(xg_ref, sims_ref, mask_ref, qw_ref, kw_ref, vw_ref, bias_ref, o_ref):
    T = G_SP * TOPK
    xb = xg_ref[...]
    xbT = xb.T
    q = jnp.dot(xb, qw_ref[...], preferred_element_type=jnp.float32) + bias_ref[0]
    kt = (jnp.dot(kw_ref[...], xbT, preferred_element_type=jnp.float32)
          + bias_ref[1][:, None])
    v = jnp.dot(xb, vw_ref[...], preferred_element_type=jnp.float32) + bias_ref[2]
    sims_col = sims_ref[0, 0][:, None]
    mask_row = mask_ref[0, 0][None, :]
    rblk = lax.broadcasted_iota(jnp.int32, (T, T), 0) // TOPK
    cblk = lax.broadcasted_iota(jnp.int32, (T, T), 1) // TOPK
    keep = jnp.logical_and(rblk == cblk, mask_row > 0)
    vw_tok = sims_col * v
    outs = []
    for h in range(NUM_HEADS):
        hs = slice(h * HEAD_DIM, (h + 1) * HEAD_DIM)
        qh = q[:, hs]
        kth = kt[hs, :]
        qq = jnp.sum(qh * qh, axis=1, keepdims=True)
        kk = jnp.sum(kth * kth, axis=0, keepdims=True)
        gr = jnp.dot(qh, kth, preferred_element_type=jnp.float32)
        d2 = qq + kk - 2.0 * gr
        dist = jnp.sqrt(jnp.maximum(d2, 1e-12))
        a = jnp.where(keep, -SCALE_F * dist, -1e9)
        a = a - jnp.max(a, axis=1, keepdims=True)
        e = jnp.exp(a)
        p = e / jnp.sum(e, axis=1, keepdims=True)
        outs.append(jnp.dot(p, vw_tok[:, hs], preferred_element_type=jnp.float32))
    o_ref[...] = sims_col * jnp.concatenate(outs, axis=1)


def _attention(xg, sims2, mask2, qw_t, kw_t, vw_t, bias):
    ntile = TOPK * G_SP
    return pl.pallas_call(
        _attn_body,
        grid=(NTOK // ntile,),
        in_specs=[
            pl.BlockSpec((ntile, C), lambda i: (i, 0)),
            pl.BlockSpec((1, 1, ntile), lambda i: (i, 0, 0)),
            pl.BlockSpec((1, 1, ntile), lambda i: (i, 0, 0)),
            pl.BlockSpec((C, C), lambda i: (0, 0)),
            pl.BlockSpec((C, C), lambda i: (0, 0)),
            pl.BlockSpec((C, C), lambda i: (0, 0)),
            pl.BlockSpec((8, C), lambda i: (0, 0)),
        ],
        out_specs=pl.BlockSpec((ntile, C), lambda i: (i, 0)),
        out_shape=jax.ShapeDtypeStruct((NTOK, C), jnp.float32),
    )(xg, sims2, mask2, qw_t, kw_t, vw_t, bias)


def _merge_body(xn_ref, acc_ref, cnt_ref, vw_ref, vb_ref, o_ref):
    xb = xn_ref[0]
    v = jnp.dot(xb, vw_ref[...], preferred_element_type=jnp.float32) + vb_ref[0]
    cnt = cnt_ref[0]
    mean = acc_ref[0] / jnp.maximum(cnt, 1.0)
    sel = jnp.where(cnt >= 0.5, mean, v)
    o_ref[0] = sel.T


def _merge(xn_t, acc, cnt3, vw_t, vb):
    return pl.pallas_call(
        _merge_body,
        grid=(B, HW // TP),
        in_specs=[
            pl.BlockSpec((1, TP, C), lambda b, i: (b, i, 0)),
            pl.BlockSpec((1, TP, C), lambda b, i: (b, i, 0)),
            pl.BlockSpec((1, TP, 1), lambda b, i: (b, i, 0)),
            pl.BlockSpec((C, C), lambda b, i: (0, 0)),
            pl.BlockSpec((1, C), lambda b, i: (0, 0)),
        ],
        out_specs=pl.BlockSpec((1, C, TP), lambda b, i: (b, 0, i)),
        out_shape=jax.ShapeDtypeStruct((B, C, HW), jnp.float32),
    )(xn_t, acc, cnt3, vw_t, vb)


def kernel(x, sims, mask, ln_w, ln_b, q_w, k_w, v_w, indices, labels, num_spixels):
    qw_t = (q_w * ln_w[None, :]).T
    kw_eff = k_w * ln_w[None, :]
    vw_t = (v_w * ln_w[None, :]).T
    qb = q_w @ ln_b
    kb = k_w @ ln_b
    vb = v_w @ ln_b
    bias = jnp.zeros((8, C), jnp.float32).at[0].set(qb).at[1].set(kb).at[2].set(vb)

    x3 = x.reshape(B, C, HW)
    xn_t = _ln_transpose(x3)

    idx_g = (indices.reshape(B, K_SP * TOPK)
             + (jnp.arange(B, dtype=jnp.int32) * HW)[:, None]).reshape(-1)
    xn_flat = xn_t.reshape(B * HW, C)
    xg = jnp.take(xn_flat, idx_g, axis=0)

    ntile = G_SP * TOPK
    sims2 = sims.reshape(NTOK // ntile, 1, ntile)
    mask2 = mask.reshape(NTOK // ntile, 1, ntile)
    out_tok = _attention(xg, sims2, mask2, qw_t, kw_eff, vw_t, bias)

    acc = jax.ops.segment_sum(out_tok, idx_g, num_segments=B * HW)
    cnt = jax.ops.segment_sum(jnp.ones((NTOK,), jnp.float32), idx_g,
                              num_segments=B * HW)
    acc = acc.reshape(B, HW, C)
    cnt3 = cnt.reshape(B, HW, 1)

    out = _merge(xn_t, acc, cnt3, vw_t, vb[None, :])
    return out.reshape(B, C, H, W)
